# baseline (speedup 1.0000x reference)
import jax
import jax.numpy as jnp
from jax import lax
from jax.experimental import pallas as pl
from jax.experimental.pallas import tpu as pltpu
from jax.experimental.pallas import tpu_sc as plsc

_INFO = plsc.get_sparse_core_info()
_NS = _INFO.num_subcores
_NC = _INFO.num_cores
_NW = _NS * _NC

_R = 4
_C = 8192
_CT = _C // 128
_CT_PER_W = _CT // _NW


def _gather_body(x_hbm, router_hbm, out_hbm, idx_v, vals_v,
                 ssems, gsems, osem):
    wid = lax.axis_index("s") * _NC + lax.axis_index("c")

    stages = []
    for t in range(_CT_PER_W):
        col = (wid * _CT_PER_W + t) * 128
        stages.append(
            pltpu.async_copy(x_hbm.at[pl.ds(0, _R), pl.ds(col, 128)],
                             idx_v.at[pl.ds(t * _R, _R), :], ssems.at[t])
        )

    gathers = [[] for _ in range(_CT_PER_W)]
    for t in range(_CT_PER_W):
        stages[t].wait()
        for r in range(_R):
            j = t * _R + r
            gathers[t].append(
                pltpu.async_copy(router_hbm.at[idx_v.at[j]], vals_v.at[j],
                                 gsems.at[t])
            )

    stores = []
    for t in range(_CT_PER_W):
        for c in gathers[t]:
            c.wait()
        col = (wid * _CT_PER_W + t) * 128
        stores.append(
            pltpu.async_copy(vals_v.at[pl.ds(t * _R, _R), :],
                             out_hbm.at[pl.ds(0, _R), pl.ds(col, 128)], osem)
        )
    for c in stores:
        c.wait()


@jax.jit
def _router_gather(x, router):
    mesh = plsc.VectorSubcoreMesh(core_axis_name="c", subcore_axis_name="s")
    return pl.kernel(
        _gather_body,
        out_type=jax.ShapeDtypeStruct((_R, _C), jnp.int32),
        mesh=mesh,
        scratch_types=[
            pltpu.VMEM((_CT_PER_W * _R, 128), jnp.int32),
            pltpu.VMEM((_CT_PER_W * _R, 128), jnp.int32),
            pltpu.SemaphoreType.DMA((_CT_PER_W,)),
            pltpu.SemaphoreType.DMA((_CT_PER_W,)),
            pltpu.SemaphoreType.DMA,
        ],
        compiler_params=pltpu.CompilerParams(use_tc_tiling_on_sc=True),
    )(x, router)


def kernel(x, router):
    return _router_gather(x, router)


# speedup vs baseline: 1.0218x; 1.0218x over previous
import jax
import jax.numpy as jnp
from jax import lax
from jax.experimental import pallas as pl
from jax.experimental.pallas import tpu as pltpu
from jax.experimental.pallas import tpu_sc as plsc

_INFO = plsc.get_sparse_core_info()
_NS = _INFO.num_subcores
_NW = _NS

_R = 4
_C = 8192
_CT = _C // 128
_CT_PER_W = _CT // _NW


def _gather_body(x_hbm, router_hbm, out_hbm, idx_v, vals_v,
                 ssems, gsems, osem):
    wid = lax.axis_index("s")

    stages = []
    for t in range(_CT_PER_W):
        col = (wid * _CT_PER_W + t) * 128
        stages.append(
            pltpu.async_copy(x_hbm.at[pl.ds(0, _R), pl.ds(col, 128)],
                             idx_v.at[pl.ds(t * _R, _R), :], ssems.at[t])
        )

    gathers = [[] for _ in range(_CT_PER_W)]
    for t in range(_CT_PER_W):
        stages[t].wait()
        for r in range(_R):
            j = t * _R + r
            gathers[t].append(
                pltpu.async_copy(router_hbm.at[idx_v.at[j]], vals_v.at[j],
                                 gsems.at[t])
            )

    stores = []
    for t in range(_CT_PER_W):
        for c in gathers[t]:
            c.wait()
        col = (wid * _CT_PER_W + t) * 128
        stores.append(
            pltpu.async_copy(vals_v.at[pl.ds(t * _R, _R), :],
                             out_hbm.at[pl.ds(0, _R), pl.ds(col, 128)], osem)
        )
    for c in stores:
        c.wait()


@jax.jit
def _router_gather(x, router):
    mesh = plsc.VectorSubcoreMesh(core_axis_name="c", subcore_axis_name="s",
                                  num_cores=1)
    return pl.kernel(
        _gather_body,
        out_type=jax.ShapeDtypeStruct((_R, _C), jnp.int32),
        mesh=mesh,
        scratch_types=[
            pltpu.VMEM((_CT_PER_W * _R, 128), jnp.int32),
            pltpu.VMEM((_CT_PER_W * _R, 128), jnp.int32),
            pltpu.SemaphoreType.DMA((_CT_PER_W,)),
            pltpu.SemaphoreType.DMA((_CT_PER_W,)),
            pltpu.SemaphoreType.DMA,
        ],
        compiler_params=pltpu.CompilerParams(use_tc_tiling_on_sc=True),
    )(x, router)


def kernel(x, router):
    return _router_gather(x, router)


# speedup vs baseline: 1.0234x; 1.0015x over previous
import jax
import jax.numpy as jnp
from jax import lax
from jax.experimental import pallas as pl
from jax.experimental.pallas import tpu as pltpu
from jax.experimental.pallas import tpu_sc as plsc

_INFO = plsc.get_sparse_core_info()
_NS = _INFO.num_subcores
_NW = _NS

_R = 4
_C = 8192
_CT = _C // 128
_CT_PER_W = _CT // _NW


def _gather_body(x_hbm, router_hbm, out_hbm, idx_v, vals_v,
                 ssems, gsems, osem):
    wid = lax.axis_index("s")

    stages = []
    for t in range(_CT_PER_W):
        col = (wid * _CT_PER_W + t) * 128
        stages.append(
            pltpu.async_copy(x_hbm.at[pl.ds(0, _R), pl.ds(col, 128)],
                             idx_v.at[pl.ds(t * _R, _R), :], ssems.at[t])
        )

    gathers = [[] for _ in range(_CT_PER_W)]
    for t in range(_CT_PER_W):
        stages[t].wait()
        for r in range(_R):
            j = t * _R + r
            gathers[t].append(
                pltpu.async_copy(router_hbm.at[idx_v.at[j]], vals_v.at[j],
                                 gsems.at[t])
            )

    stores = []
    for t in range(_CT_PER_W):
        col = (wid * _CT_PER_W + t) * 128
        for r in range(_R):
            gathers[t][r].wait()
            stores.append(
                pltpu.async_copy(vals_v.at[t * _R + r],
                                 out_hbm.at[r, pl.ds(col, 128)], osem)
            )
    for c in stores:
        c.wait()


@jax.jit
def _router_gather(x, router):
    mesh = plsc.VectorSubcoreMesh(core_axis_name="c", subcore_axis_name="s",
                                  num_cores=1)
    return pl.kernel(
        _gather_body,
        out_type=jax.ShapeDtypeStruct((_R, _C), jnp.int32),
        mesh=mesh,
        scratch_types=[
            pltpu.VMEM((_CT_PER_W * _R, 128), jnp.int32),
            pltpu.VMEM((_CT_PER_W * _R, 128), jnp.int32),
            pltpu.SemaphoreType.DMA((_CT_PER_W,)),
            pltpu.SemaphoreType.DMA((_CT_PER_W,)),
            pltpu.SemaphoreType.DMA,
        ],
        compiler_params=pltpu.CompilerParams(use_tc_tiling_on_sc=True),
    )(x, router)


def kernel(x, router):
    return _router_gather(x, router)
